# baseline (speedup 1.0000x reference)
import functools

import jax
import jax.numpy as jnp
from jax.experimental import pallas as pl
from jax.experimental.pallas import tpu as pltpu

_N_OFFSETS = 72
_N_STRIPS = _N_OFFSETS - 1
_P = 1024
_L = 128


def _nms_kernel(xs_ref, xst_ref, st_ref, en_ref, stt_ref, ent_ref, t_ref,
                keep_ref, B_ref, v_ref, vt_ref, n_rows: int):
    kio = jax.lax.broadcasted_iota(jnp.int32, (_P, _L), 1).astype(jnp.float32)
    st = st_ref[:, :]
    en = en_ref[:, :]
    v_ref[:, :] = jnp.where((kio >= st) & (kio <= en), 1.0, 0.0)

    kio_t = jax.lax.broadcasted_iota(jnp.int32, (_L, _P), 0).astype(jnp.float32)
    stt = stt_ref[:, :]
    ent = ent_ref[:, :]
    vt_ref[:, :] = jnp.where((kio_t >= stt) & (kio_t <= ent), 1.0, 0.0)

    B_ref[:, :] = jnp.zeros((_P, _P), jnp.float32)
    for k in range(_N_OFFSETS):
        vc = v_ref[:, k:k + 1]
        vr = vt_ref[k:k + 1, :]
        uc = xs_ref[:, k:k + 1] * vc
        ur = xst_ref[k:k + 1, :] * vr
        B_ref[:, :] += jnp.abs(uc * vr - vc * ur)

    t = t_ref[0, 0]
    for rb in range(_P // 128):
        rows = slice(rb * 128, (rb + 1) * 128)
        cnt = jnp.dot(v_ref[rows, :], vt_ref[:, :],
                      preferred_element_type=jnp.float32)
        dsum = B_ref[rows, :]
        dist = jnp.where(cnt > 0, dsum / jnp.maximum(cnt, 1.0), jnp.inf)
        B_ref[rows, :] = jnp.where(dist < t, 1.0, 0.0)

    lane = jax.lax.broadcasted_iota(jnp.int32, (1, _P), 1).astype(jnp.float32)

    def body(i, keep):
        fi = i.astype(jnp.float32)
        keep_i = jnp.sum(jnp.where(lane == fi, keep, 0.0))
        row = B_ref[pl.ds(i, 1), :]
        sup = jnp.where(lane > fi, row, 0.0)
        return keep * (1.0 - keep_i * sup)

    keep = jax.lax.fori_loop(0, n_rows, body, jnp.ones((1, _P), jnp.float32))
    keep_ref[:, :] = keep


def kernel(proposals, scores, nms_thres, nms_topk):
    N = proposals.shape[0]
    order = jnp.argsort(-scores)
    ps = proposals[order]

    starts = jnp.clip(jnp.round(ps[:, 2] * _N_STRIPS).astype(jnp.int32),
                      0, _N_STRIPS)
    lengths = jnp.clip(jnp.round(ps[:, 4]).astype(jnp.int32), 1, _N_OFFSETS)
    ends = jnp.clip(starts + lengths - 1, 0, _N_STRIPS)
    xs = ps[:, 5:5 + _N_OFFSETS]

    xs_p = jnp.zeros((_P, _L), jnp.float32).at[:N, :_N_OFFSETS].set(xs)
    st_p = jnp.full((_P, 1), 1e9, jnp.float32).at[:N, 0].set(
        starts.astype(jnp.float32))
    en_p = jnp.full((_P, 1), -1e9, jnp.float32).at[:N, 0].set(
        ends.astype(jnp.float32))
    t = jnp.full((1, 1), nms_thres, jnp.float32)

    keep = pl.pallas_call(
        functools.partial(_nms_kernel, n_rows=N),
        out_shape=jax.ShapeDtypeStruct((1, _P), jnp.float32),
        scratch_shapes=[
            pltpu.VMEM((_P, _P), jnp.float32),
            pltpu.VMEM((_P, _L), jnp.float32),
            pltpu.VMEM((_L, _P), jnp.float32),
        ],
    )(xs_p, xs_p.T, st_p, en_p, st_p.T, en_p.T, t)

    keep_sorted = keep[0, :N] > 0.5
    kept_scores_sorted = jnp.where(keep_sorted, scores[order], -jnp.inf)
    top_vals, top_pos = jax.lax.top_k(kept_scores_sorted, 100)
    top_idx = order[top_pos]
    num_kept = jnp.minimum(keep_sorted.sum(), nms_topk)
    return proposals[top_idx], top_vals, top_idx, num_kept


# speedup vs baseline: 1.0120x; 1.0120x over previous
import functools

import jax
import jax.numpy as jnp
from jax.experimental import pallas as pl
from jax.experimental.pallas import tpu as pltpu

_N_OFFSETS = 72
_N_STRIPS = _N_OFFSETS - 1
_P = 1024
_L = 128


def _nms_kernel(xs_ref, xst_ref, st_ref, en_ref, stt_ref, ent_ref, t_ref,
                keep_ref, B_ref, v_ref, vt_ref, n_rows: int):
    kio = jax.lax.broadcasted_iota(jnp.int32, (_P, _L), 1).astype(jnp.float32)
    st = st_ref[:, :]
    en = en_ref[:, :]
    v_ref[:, :] = jnp.where((kio >= st) & (kio <= en), 1.0, 0.0)

    kio_t = jax.lax.broadcasted_iota(jnp.int32, (_L, _P), 0).astype(jnp.float32)
    stt = stt_ref[:, :]
    ent = ent_ref[:, :]
    vt_ref[:, :] = jnp.where((kio_t >= stt) & (kio_t <= ent), 1.0, 0.0)

    B_ref[:, :] = jnp.zeros((_P, _P), jnp.float32)
    for k in range(_N_OFFSETS):
        vc = v_ref[:, k:k + 1]
        vr = vt_ref[k:k + 1, :]
        uc = xs_ref[:, k:k + 1] * vc
        ur = xst_ref[k:k + 1, :] * vr
        B_ref[:, :] += jnp.abs(uc * vr - vc * ur)

    t = t_ref[0, 0]
    col_io = jax.lax.broadcasted_iota(jnp.int32, (128, _P), 1)
    row_io = jax.lax.broadcasted_iota(jnp.int32, (128, _P), 0)
    for rb in range(_P // 128):
        rows = slice(rb * 128, (rb + 1) * 128)
        cnt = jnp.dot(v_ref[rows, :], vt_ref[:, :],
                      preferred_element_type=jnp.float32)
        dsum = B_ref[rows, :]
        dist = jnp.where(cnt > 0, dsum / jnp.maximum(cnt, 1.0), jnp.inf)
        upper = col_io > (row_io + rb * 128)
        B_ref[rows, :] = jnp.where((dist < t) & upper, 1.0, 0.0)

    lane = jax.lax.broadcasted_iota(jnp.int32, (1, _P), 1).astype(jnp.float32)

    def body(m, keep):
        i0 = m * 2
        fa = i0.astype(jnp.float32)
        keep_a = jnp.sum(jnp.where(lane == fa, keep, 0.0))
        rowa = B_ref[pl.ds(i0, 1), :]
        keep = keep * (1.0 - keep_a * rowa)
        fb = fa + 1.0
        keep_b = jnp.sum(jnp.where(lane == fb, keep, 0.0))
        rowb = B_ref[pl.ds(i0 + 1, 1), :]
        return keep * (1.0 - keep_b * rowb)

    keep = jax.lax.fori_loop(0, (n_rows + 1) // 2, body,
                             jnp.ones((1, _P), jnp.float32))
    keep_ref[:, :] = keep


def kernel(proposals, scores, nms_thres, nms_topk):
    N = proposals.shape[0]
    order = jnp.argsort(-scores)
    ps = proposals[order]

    starts = jnp.clip(jnp.round(ps[:, 2] * _N_STRIPS).astype(jnp.int32),
                      0, _N_STRIPS)
    lengths = jnp.clip(jnp.round(ps[:, 4]).astype(jnp.int32), 1, _N_OFFSETS)
    ends = jnp.clip(starts + lengths - 1, 0, _N_STRIPS)
    xs = ps[:, 5:5 + _N_OFFSETS]

    xs_p = jnp.zeros((_P, _L), jnp.float32).at[:N, :_N_OFFSETS].set(xs)
    st_p = jnp.full((_P, 1), 1e9, jnp.float32).at[:N, 0].set(
        starts.astype(jnp.float32))
    en_p = jnp.full((_P, 1), -1e9, jnp.float32).at[:N, 0].set(
        ends.astype(jnp.float32))
    t = jnp.full((1, 1), nms_thres, jnp.float32)

    keep = pl.pallas_call(
        functools.partial(_nms_kernel, n_rows=N),
        out_shape=jax.ShapeDtypeStruct((1, _P), jnp.float32),
        scratch_shapes=[
            pltpu.VMEM((_P, _P), jnp.float32),
            pltpu.VMEM((_P, _L), jnp.float32),
            pltpu.VMEM((_L, _P), jnp.float32),
        ],
    )(xs_p, xs_p.T, st_p, en_p, st_p.T, en_p.T, t)

    keep_sorted = keep[0, :N] > 0.5
    kept_scores_sorted = jnp.where(keep_sorted, scores[order], -jnp.inf)
    top_vals, top_pos = jax.lax.top_k(kept_scores_sorted, 100)
    top_idx = order[top_pos]
    num_kept = jnp.minimum(keep_sorted.sum(), nms_topk)
    return proposals[top_idx], top_vals, top_idx, num_kept


# speedup vs baseline: 1.0399x; 1.0275x over previous
import functools

import jax
import jax.numpy as jnp
from jax.experimental import pallas as pl
from jax.experimental.pallas import tpu as pltpu

_N_OFFSETS = 72
_N_STRIPS = _N_OFFSETS - 1
_P = 1024
_L = 128


def _nms_kernel(xs_ref, xst_ref, st_ref, en_ref, stt_ref, ent_ref, t_ref,
                keep_ref, B_ref, v_ref, vt_ref, n_rows: int):
    kio = jax.lax.broadcasted_iota(jnp.int32, (_P, _L), 1).astype(jnp.float32)
    st = st_ref[:, :]
    en = en_ref[:, :]
    v_ref[:, :] = jnp.where((kio >= st) & (kio <= en), 1.0, 0.0)

    kio_t = jax.lax.broadcasted_iota(jnp.int32, (_L, _P), 0).astype(jnp.float32)
    stt = stt_ref[:, :]
    ent = ent_ref[:, :]
    vt_ref[:, :] = jnp.where((kio_t >= stt) & (kio_t <= ent), 1.0, 0.0)

    B_ref[:, :] = jnp.zeros((_P, _P), jnp.float32)
    for k in range(_N_OFFSETS):
        vc = v_ref[:, k:k + 1]
        vr = vt_ref[k:k + 1, :]
        uc = xs_ref[:, k:k + 1] * vc
        ur = xst_ref[k:k + 1, :] * vr
        B_ref[:, :] += jnp.abs(uc * vr - vc * ur)

    t = t_ref[0, 0]
    col_io = jax.lax.broadcasted_iota(jnp.int32, (128, _P), 1)
    row_io = jax.lax.broadcasted_iota(jnp.int32, (128, _P), 0)
    for rb in range(_P // 128):
        rows = slice(rb * 128, (rb + 1) * 128)
        cnt = jnp.dot(v_ref[rows, :], vt_ref[:, :],
                      preferred_element_type=jnp.float32)
        dsum = B_ref[rows, :]
        dist = jnp.where(cnt > 0, dsum / jnp.maximum(cnt, 1.0), jnp.inf)
        upper = col_io > (row_io + rb * 128)
        B_ref[rows, :] = jnp.where((dist < t) & upper, 1.0, 0.0)

    lane128 = jax.lax.broadcasted_iota(jnp.int32, (1, 128), 1).astype(
        jnp.float32)
    segs = tuple(jnp.ones((1, 128), jnp.float32) for _ in range(_P // 128))
    for b in range(_P // 128):
        cs = b * 128
        if cs >= n_rows:
            break

        def body(r, segs_t, b=b, cs=cs):
            segs_l = list(segs_t)
            fi = r.astype(jnp.float32)
            keep_i = jnp.sum(jnp.where(lane128 == fi, segs_l[b], 0.0))
            row = B_ref[pl.ds(cs + r, 1), :]
            for sb in range(b, _P // 128):
                seg = row[:, sb * 128:(sb + 1) * 128]
                segs_l[sb] = segs_l[sb] * (1.0 - keep_i * seg)
            return tuple(segs_l)

        segs = jax.lax.fori_loop(0, min(128, n_rows - cs), body, segs)
    keep_ref[:, :] = jnp.concatenate(list(segs), axis=1)


def kernel(proposals, scores, nms_thres, nms_topk):
    N = proposals.shape[0]
    order = jnp.argsort(-scores)
    ps = proposals[order]

    starts = jnp.clip(jnp.round(ps[:, 2] * _N_STRIPS).astype(jnp.int32),
                      0, _N_STRIPS)
    lengths = jnp.clip(jnp.round(ps[:, 4]).astype(jnp.int32), 1, _N_OFFSETS)
    ends = jnp.clip(starts + lengths - 1, 0, _N_STRIPS)
    xs = ps[:, 5:5 + _N_OFFSETS]

    xs_p = jnp.zeros((_P, _L), jnp.float32).at[:N, :_N_OFFSETS].set(xs)
    st_p = jnp.full((_P, 1), 1e9, jnp.float32).at[:N, 0].set(
        starts.astype(jnp.float32))
    en_p = jnp.full((_P, 1), -1e9, jnp.float32).at[:N, 0].set(
        ends.astype(jnp.float32))
    t = jnp.full((1, 1), nms_thres, jnp.float32)

    keep = pl.pallas_call(
        functools.partial(_nms_kernel, n_rows=N),
        out_shape=jax.ShapeDtypeStruct((1, _P), jnp.float32),
        scratch_shapes=[
            pltpu.VMEM((_P, _P), jnp.float32),
            pltpu.VMEM((_P, _L), jnp.float32),
            pltpu.VMEM((_L, _P), jnp.float32),
        ],
    )(xs_p, xs_p.T, st_p, en_p, st_p.T, en_p.T, t)

    keep_sorted = keep[0, :N] > 0.5
    kept_scores_sorted = jnp.where(keep_sorted, scores[order], -jnp.inf)
    top_vals, top_pos = jax.lax.top_k(kept_scores_sorted, 100)
    top_idx = order[top_pos]
    num_kept = jnp.minimum(keep_sorted.sum(), nms_topk)
    return proposals[top_idx], top_vals, top_idx, num_kept


# speedup vs baseline: 1.0483x; 1.0081x over previous
import functools

import jax
import jax.numpy as jnp
from jax.experimental import pallas as pl
from jax.experimental.pallas import tpu as pltpu

_N_OFFSETS = 72
_N_STRIPS = _N_OFFSETS - 1
_P = 1024
_L = 128


def _nms_kernel(xs_ref, xst_ref, st_ref, en_ref, stt_ref, ent_ref, t_ref,
                keep_ref, B_ref, v_ref, vt_ref, n_rows: int):
    kio = jax.lax.broadcasted_iota(jnp.int32, (_P, _L), 1).astype(jnp.float32)
    st = st_ref[:, :]
    en = en_ref[:, :]
    v_ref[:, :] = jnp.where((kio >= st) & (kio <= en), 1.0, 0.0)

    kio_t = jax.lax.broadcasted_iota(jnp.int32, (_L, _P), 0).astype(jnp.float32)
    stt = stt_ref[:, :]
    ent = ent_ref[:, :]
    vt_ref[:, :] = jnp.where((kio_t >= stt) & (kio_t <= ent), 1.0, 0.0)

    B_ref[:, :] = jnp.zeros((_P, _P), jnp.float32)
    for k in range(_N_OFFSETS):
        vc = v_ref[:, k:k + 1]
        vr = vt_ref[k:k + 1, :]
        uc = xs_ref[:, k:k + 1] * vc
        ur = xst_ref[k:k + 1, :] * vr
        B_ref[:, :] += jnp.abs(uc * vr - vc * ur)

    t = t_ref[0, 0]
    col_io = jax.lax.broadcasted_iota(jnp.int32, (128, _P), 1)
    row_io = jax.lax.broadcasted_iota(jnp.int32, (128, _P), 0)
    for rb in range(_P // 128):
        rows = slice(rb * 128, (rb + 1) * 128)
        cnt = jnp.dot(v_ref[rows, :], vt_ref[:, :],
                      preferred_element_type=jnp.float32)
        dsum = B_ref[rows, :]
        dist = jnp.where(cnt > 0, dsum / jnp.maximum(cnt, 1.0), jnp.inf)
        upper = col_io > (row_io + rb * 128)
        B_ref[rows, :] = jnp.where((dist < t) & upper, 1.0, 0.0)

    lane128 = jax.lax.broadcasted_iota(jnp.int32, (1, 128), 1).astype(
        jnp.float32)
    segs = tuple(jnp.ones((1, 128), jnp.float32) for _ in range(_P // 128))
    for b in range(_P // 128):
        cs = b * 128
        if cs >= n_rows:
            break

        def body(m, segs_t, b=b, cs=cs):
            segs_l = list(segs_t)
            for half in range(2):
                r = m * 2 + half
                fi = r.astype(jnp.float32)
                keep_i = jnp.sum(jnp.where(lane128 == fi, segs_l[b], 0.0))
                row = B_ref[pl.ds(cs + r, 1), :]
                for sb in range(b, _P // 128):
                    seg = row[:, sb * 128:(sb + 1) * 128]
                    segs_l[sb] = segs_l[sb] * (1.0 - keep_i * seg)
            return tuple(segs_l)

        segs = jax.lax.fori_loop(0, (min(128, n_rows - cs) + 1) // 2, body,
                                 segs)
    keep_ref[:, :] = jnp.concatenate(list(segs), axis=1)


def kernel(proposals, scores, nms_thres, nms_topk):
    N = proposals.shape[0]
    order = jnp.argsort(-scores)
    ps = proposals[order]

    starts = jnp.clip(jnp.round(ps[:, 2] * _N_STRIPS).astype(jnp.int32),
                      0, _N_STRIPS)
    lengths = jnp.clip(jnp.round(ps[:, 4]).astype(jnp.int32), 1, _N_OFFSETS)
    ends = jnp.clip(starts + lengths - 1, 0, _N_STRIPS)
    xs = ps[:, 5:5 + _N_OFFSETS]

    xs_p = jnp.zeros((_P, _L), jnp.float32).at[:N, :_N_OFFSETS].set(xs)
    st_p = jnp.full((_P, 1), 1e9, jnp.float32).at[:N, 0].set(
        starts.astype(jnp.float32))
    en_p = jnp.full((_P, 1), -1e9, jnp.float32).at[:N, 0].set(
        ends.astype(jnp.float32))
    t = jnp.full((1, 1), nms_thres, jnp.float32)

    keep = pl.pallas_call(
        functools.partial(_nms_kernel, n_rows=N),
        out_shape=jax.ShapeDtypeStruct((1, _P), jnp.float32),
        scratch_shapes=[
            pltpu.VMEM((_P, _P), jnp.float32),
            pltpu.VMEM((_P, _L), jnp.float32),
            pltpu.VMEM((_L, _P), jnp.float32),
        ],
    )(xs_p, xs_p.T, st_p, en_p, st_p.T, en_p.T, t)

    keep_sorted = keep[0, :N] > 0.5
    kept_scores_sorted = jnp.where(keep_sorted, scores[order], -jnp.inf)
    top_vals, top_pos = jax.lax.top_k(kept_scores_sorted, 100)
    top_idx = order[top_pos]
    num_kept = jnp.minimum(keep_sorted.sum(), nms_topk)
    return proposals[top_idx], top_vals, top_idx, num_kept


# speedup vs baseline: 1.0539x; 1.0053x over previous
import functools

import jax
import jax.numpy as jnp
from jax.experimental import pallas as pl
from jax.experimental.pallas import tpu as pltpu

_N_OFFSETS = 72
_N_STRIPS = _N_OFFSETS - 1
_P = 1024
_L = 128


def _nms_kernel(xs_ref, xst_ref, st_ref, en_ref, stt_ref, ent_ref, t_ref,
                keep_ref, B_ref, v_ref, vt_ref, n_rows: int):
    kio = jax.lax.broadcasted_iota(jnp.int32, (_P, _L), 1).astype(jnp.float32)
    st = st_ref[:, :]
    en = en_ref[:, :]
    v_ref[:, :] = jnp.where((kio >= st) & (kio <= en), 1.0, 0.0)

    kio_t = jax.lax.broadcasted_iota(jnp.int32, (_L, _P), 0).astype(jnp.float32)
    stt = stt_ref[:, :]
    ent = ent_ref[:, :]
    vt_ref[:, :] = jnp.where((kio_t >= stt) & (kio_t <= ent), 1.0, 0.0)

    B_ref[:, :] = jnp.zeros((_P, _P), jnp.float32)
    for k in range(_N_OFFSETS):
        vc = v_ref[:, k:k + 1]
        vr = vt_ref[k:k + 1, :]
        uc = xs_ref[:, k:k + 1] * vc
        ur = xst_ref[k:k + 1, :] * vr
        B_ref[:, :] += jnp.abs(uc * vr - vc * ur)

    t = t_ref[0, 0]
    col_io = jax.lax.broadcasted_iota(jnp.int32, (128, _P), 1)
    row_io = jax.lax.broadcasted_iota(jnp.int32, (128, _P), 0)
    for rb in range(_P // 128):
        rows = slice(rb * 128, (rb + 1) * 128)
        cnt = jnp.dot(v_ref[rows, :], vt_ref[:, :],
                      preferred_element_type=jnp.float32)
        dsum = B_ref[rows, :]
        dist = jnp.where(cnt > 0, dsum / jnp.maximum(cnt, 1.0), jnp.inf)
        upper = col_io > (row_io + rb * 128)
        B_ref[rows, :] = jnp.where((dist < t) & upper, 1.0, 0.0)

    lane128 = jax.lax.broadcasted_iota(jnp.int32, (1, 128), 1).astype(
        jnp.float32)
    segs = tuple(jnp.ones((1, 128), jnp.float32) for _ in range(_P // 128))
    for b in range(_P // 128):
        cs = b * 128
        if cs >= n_rows:
            break

        def body(m, segs_t, b=b, cs=cs):
            segs_l = list(segs_t)
            for half in range(4):
                r = m * 4 + half
                fi = r.astype(jnp.float32)
                keep_i = jnp.sum(jnp.where(lane128 == fi, segs_l[b], 0.0))
                row = B_ref[pl.ds(cs + r, 1), :]
                for sb in range(b, _P // 128):
                    seg = row[:, sb * 128:(sb + 1) * 128]
                    segs_l[sb] = segs_l[sb] * (1.0 - keep_i * seg)
            return tuple(segs_l)

        segs = jax.lax.fori_loop(0, (min(128, n_rows - cs) + 3) // 4, body,
                                 segs)
    keep_ref[:, :] = jnp.concatenate(list(segs), axis=1)


def kernel(proposals, scores, nms_thres, nms_topk):
    N = proposals.shape[0]
    order = jnp.argsort(-scores)
    ps = proposals[order]

    starts = jnp.clip(jnp.round(ps[:, 2] * _N_STRIPS).astype(jnp.int32),
                      0, _N_STRIPS)
    lengths = jnp.clip(jnp.round(ps[:, 4]).astype(jnp.int32), 1, _N_OFFSETS)
    ends = jnp.clip(starts + lengths - 1, 0, _N_STRIPS)
    xs = ps[:, 5:5 + _N_OFFSETS]

    xs_p = jnp.zeros((_P, _L), jnp.float32).at[:N, :_N_OFFSETS].set(xs)
    st_p = jnp.full((_P, 1), 1e9, jnp.float32).at[:N, 0].set(
        starts.astype(jnp.float32))
    en_p = jnp.full((_P, 1), -1e9, jnp.float32).at[:N, 0].set(
        ends.astype(jnp.float32))
    t = jnp.full((1, 1), nms_thres, jnp.float32)

    keep = pl.pallas_call(
        functools.partial(_nms_kernel, n_rows=N),
        out_shape=jax.ShapeDtypeStruct((1, _P), jnp.float32),
        scratch_shapes=[
            pltpu.VMEM((_P, _P), jnp.float32),
            pltpu.VMEM((_P, _L), jnp.float32),
            pltpu.VMEM((_L, _P), jnp.float32),
        ],
    )(xs_p, xs_p.T, st_p, en_p, st_p.T, en_p.T, t)

    keep_sorted = keep[0, :N] > 0.5
    kept_scores_sorted = jnp.where(keep_sorted, scores[order], -jnp.inf)
    top_vals, top_pos = jax.lax.top_k(kept_scores_sorted, 100)
    top_idx = order[top_pos]
    num_kept = jnp.minimum(keep_sorted.sum(), nms_topk)
    return proposals[top_idx], top_vals, top_idx, num_kept
